# Initial kernel scaffold; baseline (speedup 1.0000x reference)
import jax
import jax.numpy as jnp
from jax.experimental import pallas as pl


def kernel(user_embedding, item_embedding, beh_embeddings, prompt_emb, edge_index, edge_w, u_w, i_w):
    raise NotImplementedError("write your pallas kernel here")


# speedup vs baseline: 4.8147x; 4.8147x over previous
import functools

import jax
import jax.numpy as jnp
from jax import lax
from jax.experimental import pallas as pl
from jax.experimental.pallas import tpu as pltpu
from jax.experimental.pallas import tpu_sc as plsc

U = 50000
D = 64
B = 3
E = 800000
HALF = 32
NT = 16
EPT = 51200
EP = NT * EPT
SUP = 512
NSUP = EPT // SUP
SUB = 128
NSUB = SUP // SUB
UP = 51200
RPT = UP // NT


def _sc_body(table, src_all, dst_all, w_all, zeros_in, out,
             acc, src_v, dst_v, w_v, gbuf, sem):
    c = lax.axis_index("c")
    t = lax.axis_index("s")
    stripe = t * RPT

    pltpu.sync_copy(zeros_in, acc.at[pl.ds(stripe, RPT), :])
    plsc.subcore_barrier()

    @pl.loop(jnp.int32(0), jnp.int32(2 * B))
    def _spmm(s6):
        side = lax.div(s6, jnp.int32(B))
        b = s6 - side * B
        tbase = side * (2 * U) + c * U

        @pl.loop(jnp.int32(0), jnp.int32(NSUP))
        def _super(sc):
            pltpu.sync_copy(src_all.at[side, b, t, pl.ds(sc * SUP, SUP)], src_v)
            pltpu.sync_copy(dst_all.at[side, b, t, pl.ds(sc * NSUB, NSUB), :],
                            dst_v)
            pltpu.sync_copy(w_all.at[b, t, pl.ds(sc * SUP, SUP)], w_v)

            bvec = jnp.full((16,), tbase, dtype=jnp.int32)

            @pl.loop(jnp.int32(0), jnp.int32(SUP // 16))
            def _addbase(g):
                src_v[pl.ds(g * 16, 16)] = src_v[pl.ds(g * 16, 16)] + bvec

            cps = [
                pltpu.async_copy(table.at[src_v.at[pl.ds(j * SUB, SUB)]],
                                 gbuf.at[pl.ds(j * SUB, SUB), :], sem)
                for j in range(NSUB)
            ]
            for cp in cps:
                cp.wait()

            @pl.loop(jnp.int32(0), jnp.int32(SUP // 16))
            def _mul(g):
                for k in range(16):
                    r = g * 16 + k
                    wb = plsc.load_gather(
                        w_v, [jnp.full((16,), r, dtype=jnp.int32)])
                    gbuf[r, pl.ds(0, 16)] = gbuf[r, pl.ds(0, 16)] * wb
                    gbuf[r, pl.ds(16, 16)] = gbuf[r, pl.ds(16, 16)] * wb

            for j in range(NSUB):
                pltpu.sync_copy(gbuf.at[pl.ds(j * SUB, SUB), :],
                                acc.at[dst_v.at[jnp.int32(j)]], add=True)

        plsc.subcore_barrier()
        pltpu.sync_copy(acc.at[pl.ds(stripe, RPT), :],
                        out.at[side, b, c, pl.ds(stripe, RPT), :])
        pltpu.sync_copy(zeros_in, acc.at[pl.ds(stripe, RPT), :])
        plsc.subcore_barrier()


def _sc_spmm(table, src_all, dst_all, w_all, zeros_in):
    mesh = plsc.VectorSubcoreMesh(core_axis_name="c", subcore_axis_name="s",
                                  num_cores=2, num_subcores=NT)
    return pl.kernel(
        _sc_body,
        out_type=jax.ShapeDtypeStruct((2, B, 2, UP, HALF), jnp.float32),
        mesh=mesh,
        compiler_params=pltpu.CompilerParams(needs_layout_passes=False,
                                             use_tc_tiling_on_sc=False),
        scratch_types=[
            pltpu.VMEM_SHARED((UP, HALF), jnp.float32),
            pltpu.VMEM((SUP,), jnp.int32),
            pltpu.VMEM((NSUB, SUB), jnp.int32),
            pltpu.VMEM((SUP,), jnp.float32),
            pltpu.VMEM((SUP, HALF), jnp.float32),
            pltpu.SemaphoreType.DMA,
        ],
    )(table, src_all, dst_all, w_all, zeros_in)


def _dot(a, b):
    return jnp.dot(a, b, preferred_element_type=jnp.float32,
                   precision=lax.Precision.HIGHEST)


def _tc_body(agg_ref, w_ref, p_ref, big_ref, mean_ref):
    A = agg_ref[0]
    W = w_ref[0]
    pz = _dot(p_ref[...], W)
    W0 = W[:HALF]
    W1 = W[HALF:]
    zsum = None
    for bi in range(B):
        z = _dot(A[bi, 0], W0) + _dot(A[bi, 1], W1)
        if bi == B - 1:
            z = z + pz
        big_ref[0, bi] = jax.nn.sigmoid(z)
        zsum = z if zsum is None else zsum + z
    mean_ref[0] = jax.nn.sigmoid(zsum * (1.0 / B))


def _tc_tail(agg, Wst, prompt):
    R = 1600
    grid = (2, UP // R)
    return pl.pallas_call(
        _tc_body,
        grid=grid,
        in_specs=[
            pl.BlockSpec((1, B, 2, R, HALF), lambda s, r: (s, s * 0, s * 0, r, s * 0)),
            pl.BlockSpec((1, D, D), lambda s, r: (s, s * 0, s * 0)),
            pl.BlockSpec((1, D), lambda s, r: (s * 0, s * 0)),
        ],
        out_specs=[
            pl.BlockSpec((1, B, R, D), lambda s, r: (s, s * 0, r, s * 0)),
            pl.BlockSpec((1, R, D), lambda s, r: (s, r, s * 0)),
        ],
        out_shape=[
            jax.ShapeDtypeStruct((2, B, UP, D), jnp.float32),
            jax.ShapeDtypeStruct((2, UP, D), jnp.float32),
        ],
    )(agg, Wst, prompt)


def kernel(user_embedding, item_embedding, beh_embeddings, prompt_emb,
           edge_index, edge_w, u_w, i_w):
    f32 = jnp.float32
    item_t = item_embedding.astype(f32)
    user_t = user_embedding.astype(f32)
    item_split = item_t.reshape(U, 2, HALF).transpose(1, 0, 2)
    user_split = user_t.reshape(U, 2, HALF).transpose(1, 0, 2)
    table = jnp.concatenate([item_split, user_split], axis=0).reshape(4 * U, HALF)

    rows = edge_index[:, 0, :].astype(jnp.int32)
    cols = edge_index[:, 1, :].astype(jnp.int32)
    pad = EP - E
    rows_p = jnp.pad(rows, ((0, 0), (0, pad)))
    cols_p = jnp.pad(cols, ((0, 0), (0, pad)))
    w_p = jnp.pad(edge_w.astype(f32), ((0, 0), (0, pad)))

    src_all = jnp.stack([cols_p, rows_p]).reshape(2, B, NT, EPT)
    dst_all = jnp.stack([rows_p, cols_p]).reshape(2, B, NT, EPT // SUB, SUB)
    w_all = w_p.reshape(B, NT, EPT)
    zeros_in = jnp.zeros((RPT, HALF), f32)

    agg = _sc_spmm(table, src_all, dst_all, w_all, zeros_in)

    Wst = jnp.stack([u_w.astype(f32), i_w.astype(f32)])
    big, mean = _tc_tail(agg, Wst, prompt_emb.astype(f32))
    f64 = jnp.float64
    return (mean[0, :U].astype(f64), mean[1, :U].astype(f64),
            big[0, :, :U].astype(f64), big[1, :, :U].astype(f64))


# speedup vs baseline: 5.3684x; 1.1150x over previous
import functools

import jax
import jax.numpy as jnp
from jax import lax
from jax.experimental import pallas as pl
from jax.experimental.pallas import tpu as pltpu
from jax.experimental.pallas import tpu_sc as plsc

U = 50000
D = 64
B = 3
E = 800000
HALF = 32
NT = 16
EPT = 51200
EP = NT * EPT
SUP = 512
NSUP = EPT // SUP
SUB = 128
NSUB = SUP // SUB
UP = 51200
RPT = UP // NT


def _sc_body(table, src_all, dst_all, w_all, zeros_in, out,
             acc, src_v, dst_v, w_v, gbuf, sem):
    c = lax.axis_index("c")
    t = lax.axis_index("s")
    stripe = t * RPT

    pltpu.sync_copy(zeros_in, acc.at[pl.ds(stripe, RPT), :])
    plsc.subcore_barrier()

    @pl.loop(jnp.int32(0), jnp.int32(2 * B))
    def _spmm(s6):
        side = lax.div(s6, jnp.int32(B))
        b = s6 - side * B
        tbase = side * (2 * U) + c * U

        @pl.loop(jnp.int32(0), jnp.int32(NSUP))
        def _super(sc):
            pltpu.sync_copy(src_all.at[side, b, t, pl.ds(sc * SUP, SUP)], src_v)
            pltpu.sync_copy(dst_all.at[side, b, t, pl.ds(sc * NSUB, NSUB), :],
                            dst_v)
            pltpu.sync_copy(w_all.at[b, t, pl.ds(sc * SUP, SUP)], w_v)

            bvec = jnp.full((16,), tbase, dtype=jnp.int32)

            @pl.loop(jnp.int32(0), jnp.int32(SUP // 16))
            def _addbase(g):
                src_v[pl.ds(g * 16, 16)] = src_v[pl.ds(g * 16, 16)] + bvec

            cps = [
                pltpu.async_copy(table.at[src_v.at[pl.ds(j * SUB, SUB)]],
                                 gbuf.at[pl.ds(j * SUB, SUB), :], sem)
                for j in range(NSUB)
            ]
            for cp in cps:
                cp.wait()

            @pl.loop(jnp.int32(0), jnp.int32(SUP // 16))
            def _mul(g):
                for k in range(16):
                    r = g * 16 + k
                    wb = plsc.load_gather(
                        w_v, [jnp.full((16,), r, dtype=jnp.int32)])
                    gbuf[r, pl.ds(0, 16)] = gbuf[r, pl.ds(0, 16)] * wb
                    gbuf[r, pl.ds(16, 16)] = gbuf[r, pl.ds(16, 16)] * wb

            for j in range(NSUB):
                pltpu.sync_copy(gbuf.at[pl.ds(j * SUB, SUB), :],
                                acc.at[dst_v.at[jnp.int32(j)]], add=True)

        plsc.subcore_barrier()
        pltpu.sync_copy(acc.at[pl.ds(stripe, RPT), :],
                        out.at[side, b, c, pl.ds(stripe, RPT), :])
        pltpu.sync_copy(zeros_in, acc.at[pl.ds(stripe, RPT), :])
        plsc.subcore_barrier()


def _sc_spmm(table, src_all, dst_all, w_all, zeros_in):
    mesh = plsc.VectorSubcoreMesh(core_axis_name="c", subcore_axis_name="s",
                                  num_cores=2, num_subcores=NT)
    return pl.kernel(
        _sc_body,
        out_type=jax.ShapeDtypeStruct((2, B, 2, UP, HALF), jnp.float32),
        mesh=mesh,
        compiler_params=pltpu.CompilerParams(needs_layout_passes=False,
                                             use_tc_tiling_on_sc=False),
        scratch_types=[
            pltpu.VMEM_SHARED((UP, HALF), jnp.float32),
            pltpu.VMEM((SUP,), jnp.int32),
            pltpu.VMEM((NSUB, SUB), jnp.int32),
            pltpu.VMEM((SUP,), jnp.float32),
            pltpu.VMEM((SUP, HALF), jnp.float32),
            pltpu.SemaphoreType.DMA,
        ],
    )(table, src_all, dst_all, w_all, zeros_in)


def _dot(a, b):
    return jnp.dot(a, b, preferred_element_type=jnp.float32,
                   precision=lax.Precision.HIGHEST)


def _f64bits(y):
    bits = lax.bitcast_convert_type(y, jnp.int32)
    s = jnp.bitwise_and(bits, jnp.int32(-2147483648))
    e = jnp.bitwise_and(jnp.right_shift(bits, 23), 0xFF)
    m = jnp.bitwise_and(bits, 0x7FFFFF)
    hi = s | jnp.left_shift(e + 896, 20) | jnp.right_shift(m, 3)
    lo = jnp.left_shift(jnp.bitwise_and(m, 7), 29)
    return hi, lo


def _tc_body(agg_ref, w_ref, p_ref, bh_ref, bl_ref, mh_ref, ml_ref):
    A = agg_ref[0]
    W = w_ref[0]
    pz = _dot(p_ref[...], W)
    W0 = W[:HALF]
    W1 = W[HALF:]
    zsum = None
    for bi in range(B):
        z = _dot(A[bi, 0], W0) + _dot(A[bi, 1], W1)
        if bi == B - 1:
            z = z + pz
        hi, lo = _f64bits(jax.nn.sigmoid(z))
        bh_ref[0, bi] = hi
        bl_ref[0, bi] = lo
        zsum = z if zsum is None else zsum + z
    hi, lo = _f64bits(jax.nn.sigmoid(zsum * (1.0 / B)))
    mh_ref[0] = hi
    ml_ref[0] = lo


def _tc_tail(agg, Wst, prompt):
    R = 1600
    grid = (2, UP // R)
    return pl.pallas_call(
        _tc_body,
        grid=grid,
        in_specs=[
            pl.BlockSpec((1, B, 2, R, HALF), lambda s, r: (s, s * 0, s * 0, r, s * 0)),
            pl.BlockSpec((1, D, D), lambda s, r: (s, s * 0, s * 0)),
            pl.BlockSpec((1, D), lambda s, r: (s * 0, s * 0)),
        ],
        out_specs=[
            pl.BlockSpec((1, B, R, D), lambda s, r: (s, s * 0, r, s * 0)),
            pl.BlockSpec((1, B, R, D), lambda s, r: (s, s * 0, r, s * 0)),
            pl.BlockSpec((1, R, D), lambda s, r: (s, r, s * 0)),
            pl.BlockSpec((1, R, D), lambda s, r: (s, r, s * 0)),
        ],
        out_shape=[
            jax.ShapeDtypeStruct((2, B, UP, D), jnp.int32),
            jax.ShapeDtypeStruct((2, B, UP, D), jnp.int32),
            jax.ShapeDtypeStruct((2, UP, D), jnp.int32),
            jax.ShapeDtypeStruct((2, UP, D), jnp.int32),
        ],
    )(agg, Wst, prompt)


def kernel(user_embedding, item_embedding, beh_embeddings, prompt_emb,
           edge_index, edge_w, u_w, i_w):
    f32 = jnp.float32
    item_t = item_embedding.astype(f32)
    user_t = user_embedding.astype(f32)
    item_split = item_t.reshape(U, 2, HALF).transpose(1, 0, 2)
    user_split = user_t.reshape(U, 2, HALF).transpose(1, 0, 2)
    table = jnp.concatenate([item_split, user_split], axis=0).reshape(4 * U, HALF)

    rows = edge_index[:, 0, :].astype(jnp.int32)
    cols = edge_index[:, 1, :].astype(jnp.int32)
    pad = EP - E
    rows_p = jnp.pad(rows, ((0, 0), (0, pad)))
    cols_p = jnp.pad(cols, ((0, 0), (0, pad)))
    w_p = jnp.pad(edge_w.astype(f32), ((0, 0), (0, pad)))

    src_all = jnp.stack([cols_p, rows_p]).reshape(2, B, NT, EPT)
    dst_all = jnp.stack([rows_p, cols_p]).reshape(2, B, NT, EPT // SUB, SUB)
    w_all = w_p.reshape(B, NT, EPT)
    zeros_in = jnp.zeros((RPT, HALF), f32)

    agg = _sc_spmm(table, src_all, dst_all, w_all, zeros_in)

    Wst = jnp.stack([u_w.astype(f32), i_w.astype(f32)])
    bh, bl, mh, ml = _tc_tail(agg, Wst, prompt_emb.astype(f32))

    def comb(hi, lo):
        return lax.bitcast_convert_type(jnp.stack([lo, hi], axis=-1),
                                        jnp.float64)

    return (comb(mh[0, :U], ml[0, :U]), comb(mh[1, :U], ml[1, :U]),
            comb(bh[0, :, :U], bl[0, :, :U]), comb(bh[1, :, :U], bl[1, :, :U]))


# speedup vs baseline: 6.8122x; 1.2689x over previous
import functools

import jax
import jax.numpy as jnp
from jax import lax
from jax.experimental import pallas as pl
from jax.experimental.pallas import tpu as pltpu
from jax.experimental.pallas import tpu_sc as plsc

U = 50000
D = 64
B = 3
E = 800000
HALF = 32
NT = 16
SUP = 384
NSUP = 132
EPT = SUP * NSUP
EP = NT * EPT
SUB = 128
NSUB = SUP // SUB
UP = 50048
RPT = UP // NT


def _sc_body(table, idx_all, w_all, zeros_in, out,
             acc, comb_v, w_v, gbuf,
             gsem0, gsem1, ssem0, ssem1, isem0, isem1):
    c = lax.axis_index("c")
    t = lax.axis_index("s")
    stripe = t * RPT
    gsem = (gsem0, gsem1)
    ssem = (ssem0, ssem1)
    isem = (isem0, isem1)

    def fire_idx(side, bb, s, buf):
        bi = jnp.int32(buf)
        pltpu.async_copy(idx_all.at[side, bb, t, s], comb_v.at[bi],
                         isem[buf])
        pltpu.async_copy(w_all.at[bb, t, pl.ds(s * SUP, SUP)],
                         w_v.at[bi], isem[buf])

    def wait_idx(side, bb, s, buf):
        bi = jnp.int32(buf)
        pltpu.make_async_copy(idx_all.at[side, bb, t, s], comb_v.at[bi],
                              isem[buf]).wait()
        pltpu.make_async_copy(w_all.at[bb, t, pl.ds(s * SUP, SUP)],
                              w_v.at[bi], isem[buf]).wait()

    def add_base(buf, bvec):
        bi = jnp.int32(buf)
        for j in range(NSUB):
            ji = jnp.int32(j)

            @pl.loop(jnp.int32(0), jnp.int32(SUB // 16))
            def _ab(g):
                sl = pl.ds(g * 16, 16)
                comb_v[bi, 0, ji, sl] = comb_v[bi, 0, ji, sl] + bvec

    def fire_gathers(buf):
        bi = jnp.int32(buf)
        for j in range(NSUB):
            pltpu.async_copy(table.at[comb_v.at[bi, jnp.int32(0), jnp.int32(j)]],
                             gbuf.at[bi, pl.ds(j * SUB, SUB), :], gsem[buf])

    def wait_gathers(buf):
        bi = jnp.int32(buf)
        for j in range(NSUB):
            pltpu.make_async_copy(table.at[comb_v.at[bi, jnp.int32(0), jnp.int32(j)]],
                                  gbuf.at[bi, pl.ds(j * SUB, SUB), :],
                                  gsem[buf]).wait()

    def fire_scatters(buf):
        bi = jnp.int32(buf)
        for j in range(NSUB):
            pltpu.async_copy(gbuf.at[bi, pl.ds(j * SUB, SUB), :],
                             acc.at[comb_v.at[bi, jnp.int32(1), jnp.int32(j)]],
                             ssem[buf], add=True)

    def wait_scatters(buf):
        bi = jnp.int32(buf)
        for j in range(NSUB):
            pltpu.make_async_copy(gbuf.at[bi, pl.ds(j * SUB, SUB), :],
                                  acc.at[comb_v.at[bi, jnp.int32(1), jnp.int32(j)]],
                                  ssem[buf]).wait()

    def mul_rows(buf):
        bv = jnp.full((16,), buf, dtype=jnp.int32)
        bi = jnp.int32(buf)

        @pl.loop(jnp.int32(0), jnp.int32(SUP // 16))
        def _mul(g):
            for k in range(16):
                r = g * 16 + k
                wb = plsc.load_gather(
                    w_v, [bv, jnp.full((16,), r, dtype=jnp.int32)])
                gbuf[bi, r, pl.ds(0, 16)] = gbuf[bi, r, pl.ds(0, 16)] * wb
                gbuf[bi, r, pl.ds(16, 16)] = gbuf[bi, r, pl.ds(16, 16)] * wb

    pltpu.sync_copy(zeros_in, acc.at[pl.ds(stripe, RPT), :])
    plsc.subcore_barrier()

    @pl.loop(jnp.int32(0), jnp.int32(2 * B))
    def _spmm(s6):
        side = lax.div(s6, jnp.int32(B))
        bb = s6 - side * B
        tbase = side * (2 * U) + c * U
        bvec = jnp.full((16,), tbase, dtype=jnp.int32)

        fire_idx(side, bb, jnp.int32(0), 0)
        wait_idx(side, bb, jnp.int32(0), 0)
        add_base(0, bvec)
        fire_gathers(0)

        @pl.loop(jnp.int32(0), jnp.int32(NSUP), step=2)
        def _super(sc0):
            for bpy in range(2):
                s = sc0 + bpy
                buf, o = bpy, 1 - bpy
                wait_gathers(buf)

                @pl.when(s > 0)
                def _():
                    wait_scatters(o)

                @pl.when(s + 1 < NSUP)
                def _():
                    fire_idx(side, bb, s + 1, o)
                    wait_idx(side, bb, s + 1, o)
                    add_base(o, bvec)
                    fire_gathers(o)

                mul_rows(buf)
                fire_scatters(buf)

        wait_scatters((NSUP - 1) % 2)
        plsc.subcore_barrier()
        pltpu.sync_copy(acc.at[pl.ds(stripe, RPT), :],
                        out.at[side, bb, c, pl.ds(stripe, RPT), :])
        pltpu.sync_copy(zeros_in, acc.at[pl.ds(stripe, RPT), :])
        plsc.subcore_barrier()


def _sc_spmm(table, idx_all, w_all, zeros_in):
    mesh = plsc.VectorSubcoreMesh(core_axis_name="c", subcore_axis_name="s",
                                  num_cores=2, num_subcores=NT)
    return pl.kernel(
        _sc_body,
        out_type=jax.ShapeDtypeStruct((2, B, 2, UP, HALF), jnp.float32),
        mesh=mesh,
        compiler_params=pltpu.CompilerParams(needs_layout_passes=False,
                                             use_tc_tiling_on_sc=False),
        scratch_types=[
            pltpu.VMEM_SHARED((UP, HALF), jnp.float32),
            pltpu.VMEM((2, 2, NSUB, SUB), jnp.int32),
            pltpu.VMEM((2, SUP), jnp.float32),
            pltpu.VMEM((2, SUP, HALF), jnp.float32),
            pltpu.SemaphoreType.DMA,
            pltpu.SemaphoreType.DMA,
            pltpu.SemaphoreType.DMA,
            pltpu.SemaphoreType.DMA,
            pltpu.SemaphoreType.DMA,
            pltpu.SemaphoreType.DMA,
        ],
    )(table, idx_all, w_all, zeros_in)


def _dot(a, b):
    return jnp.dot(a, b, preferred_element_type=jnp.float32,
                   precision=lax.Precision.HIGHEST)


def _f64bits(y):
    bits = lax.bitcast_convert_type(y, jnp.int32)
    s = jnp.bitwise_and(bits, jnp.int32(-2147483648))
    e = jnp.bitwise_and(jnp.right_shift(bits, 23), 0xFF)
    m = jnp.bitwise_and(bits, 0x7FFFFF)
    hi = s | jnp.left_shift(e + 896, 20) | jnp.right_shift(m, 3)
    lo = jnp.left_shift(jnp.bitwise_and(m, 7), 29)
    return hi, lo


def _tc_body(agg_ref, w_ref, p_ref, bh_ref, bl_ref, mh_ref, ml_ref):
    A = agg_ref[0]
    W = w_ref[0]
    pz = _dot(p_ref[...], W)
    W0 = W[:HALF]
    W1 = W[HALF:]
    zsum = None
    for bi in range(B):
        z = _dot(A[bi, 0], W0) + _dot(A[bi, 1], W1)
        if bi == B - 1:
            z = z + pz
        hi, lo = _f64bits(jax.nn.sigmoid(z))
        bh_ref[0, bi] = hi
        bl_ref[0, bi] = lo
        zsum = z if zsum is None else zsum + z
    hi, lo = _f64bits(jax.nn.sigmoid(zsum * (1.0 / B)))
    mh_ref[0] = hi
    ml_ref[0] = lo


def _tc_tail(agg, Wst, prompt):
    R = 3128
    grid = (2, UP // R)
    return pl.pallas_call(
        _tc_body,
        grid=grid,
        in_specs=[
            pl.BlockSpec((1, B, 2, R, HALF), lambda s, r: (s, s * 0, s * 0, r, s * 0)),
            pl.BlockSpec((1, D, D), lambda s, r: (s, s * 0, s * 0)),
            pl.BlockSpec((1, D), lambda s, r: (s * 0, s * 0)),
        ],
        out_specs=[
            pl.BlockSpec((1, B, R, D), lambda s, r: (s, s * 0, r, s * 0)),
            pl.BlockSpec((1, B, R, D), lambda s, r: (s, s * 0, r, s * 0)),
            pl.BlockSpec((1, R, D), lambda s, r: (s, r, s * 0)),
            pl.BlockSpec((1, R, D), lambda s, r: (s, r, s * 0)),
        ],
        out_shape=[
            jax.ShapeDtypeStruct((2, B, UP, D), jnp.int32),
            jax.ShapeDtypeStruct((2, B, UP, D), jnp.int32),
            jax.ShapeDtypeStruct((2, UP, D), jnp.int32),
            jax.ShapeDtypeStruct((2, UP, D), jnp.int32),
        ],
    )(agg, Wst, prompt)


def kernel(user_embedding, item_embedding, beh_embeddings, prompt_emb,
           edge_index, edge_w, u_w, i_w):
    f32 = jnp.float32
    item_t = item_embedding.astype(f32)
    user_t = user_embedding.astype(f32)
    item_split = item_t.reshape(U, 2, HALF).transpose(1, 0, 2)
    user_split = user_t.reshape(U, 2, HALF).transpose(1, 0, 2)
    table = jnp.concatenate([item_split, user_split], axis=0).reshape(4 * U, HALF)

    rows = edge_index[:, 0, :].astype(jnp.int32)
    cols = edge_index[:, 1, :].astype(jnp.int32)
    pad = EP - E
    rows_p = jnp.pad(rows, ((0, 0), (0, pad)))
    cols_p = jnp.pad(cols, ((0, 0), (0, pad)))
    w_p = jnp.pad(edge_w.astype(f32), ((0, 0), (0, pad)))

    srcr = jnp.stack([cols_p, rows_p]).reshape(2, B, NT, NSUP, NSUB, SUB)
    dstr = jnp.stack([rows_p, cols_p]).reshape(2, B, NT, NSUP, NSUB, SUB)
    idx_all = jnp.stack([srcr, dstr], axis=4)
    w_all = w_p.reshape(B, NT, EPT)
    zeros_in = jnp.zeros((RPT, HALF), f32)

    agg = _sc_spmm(table, idx_all, w_all, zeros_in)

    Wst = jnp.stack([u_w.astype(f32), i_w.astype(f32)])
    bh, bl, mh, ml = _tc_tail(agg, Wst, prompt_emb.astype(f32))

    def comb(hi, lo):
        return lax.bitcast_convert_type(jnp.stack([lo, hi], axis=-1),
                                        jnp.float64)

    return (comb(mh[0, :U], ml[0, :U]), comb(mh[1, :U], ml[1, :U]),
            comb(bh[0, :, :U], bl[0, :, :U]), comb(bh[1, :, :U], bl[1, :, :U]))
